# Initial kernel scaffold; baseline (speedup 1.0000x reference)
import jax
import jax.numpy as jnp
from jax.experimental import pallas as pl


def kernel(x, edge_index, W1, W2):
    raise NotImplementedError("write your pallas kernel here")


# speedup vs baseline: 35.8411x; 35.8411x over previous
import functools

import jax
import jax.numpy as jnp
from jax import lax
from jax.experimental import pallas as pl
from jax.experimental.pallas import tpu as pltpu
from jax.experimental.pallas import tpu_sc as plsc

N = 10000
E = 320000
D_IN = 128
D_HID = 16
D_OUT = 40

NC = 2
NS = 16
NW = NC * NS

NPAD = 10240
RPT = NPAD // NS
CHUNK = 128
CH = -(-E // (NW * CHUNK))
EPW = CH * CHUNK
E_PAD = EPW * NW

_mesh = plsc.VectorSubcoreMesh(
    core_axis_name="c", subcore_axis_name="s", num_cores=NC, num_subcores=NS)
_sc_params = pltpu.CompilerParams(use_tc_tiling_on_sc=False)


@functools.partial(
    pl.kernel,
    out_type=jax.ShapeDtypeStruct((NC, NPAD), jnp.float32),
    mesh=_mesh,
    scratch_types=[
        pltpu.VMEM((CH, CHUNK), jnp.int32),
        pltpu.VMEM((CHUNK,), jnp.float32),
        pltpu.VMEM((RPT,), jnp.float32),
        pltpu.VMEM_SHARED((NPAD,), jnp.float32),
    ],
    compiler_params=_sc_params,
)
def _sc_degree(dst_hbm, out_hbm, idx_v, ones_v, zeros_v, table):
    cid = lax.axis_index("c")
    sid = lax.axis_index("s")
    w = sid * NC + cid

    def fill_ones(i, _):
        ones_v[pl.ds(i * 16, 16)] = jnp.ones((16,), jnp.float32)
        return 0

    lax.fori_loop(0, CHUNK // 16, fill_ones, 0)

    def fill_zeros(i, _):
        zeros_v[pl.ds(i * 16, 16)] = jnp.zeros((16,), jnp.float32)
        return 0

    lax.fori_loop(0, RPT // 16, fill_zeros, 0)

    pltpu.sync_copy(zeros_v, table.at[pl.ds(sid * RPT, RPT)])
    pltpu.sync_copy(dst_hbm.at[w], idx_v)
    plsc.subcore_barrier()

    def body(j, _):
        pltpu.sync_copy(ones_v, table.at[idx_v.at[j]], add=True)
        return 0

    lax.fori_loop(0, CH, body, 0)
    plsc.subcore_barrier()
    pltpu.sync_copy(table.at[pl.ds(sid * RPT, RPT)],
                    out_hbm.at[cid, pl.ds(sid * RPT, RPT)])


@functools.partial(
    pl.kernel,
    out_type=jax.ShapeDtypeStruct((NC, NPAD, D_HID), jnp.float32),
    mesh=_mesh,
    scratch_types=[
        pltpu.VMEM((CH, CHUNK), jnp.int32),
        pltpu.VMEM((CH, CHUNK), jnp.int32),
        pltpu.VMEM((CHUNK, D_HID), jnp.float32),
        pltpu.VMEM_SHARED((NPAD, D_HID), jnp.float32),
        pltpu.SemaphoreType.DMA,
    ],
    compiler_params=_sc_params,
)
def _sc_aggregate(src_hbm, dst_hbm, y_hbm, out_hbm,
                  src_v, dst_v, rows_v, table, sem):
    cid = lax.axis_index("c")
    sid = lax.axis_index("s")
    w = sid * NC + cid

    pltpu.sync_copy(y_hbm.at[pl.ds(sid * RPT, RPT)],
                    table.at[pl.ds(sid * RPT, RPT)])
    pltpu.sync_copy(src_hbm.at[w], src_v)
    pltpu.sync_copy(dst_hbm.at[w], dst_v)
    plsc.subcore_barrier()

    def body(j, _):
        pltpu.async_copy(y_hbm.at[src_v.at[j]], rows_v, sem).wait()
        pltpu.sync_copy(rows_v, table.at[dst_v.at[j]], add=True)
        return 0

    lax.fori_loop(0, CH, body, 0)
    plsc.subcore_barrier()
    pltpu.sync_copy(table.at[pl.ds(sid * RPT, RPT)],
                    out_hbm.at[cid, pl.ds(sid * RPT, RPT)])


def _mm_body(x_ref, w_ref, o_ref):
    o_ref[...] = jnp.dot(x_ref[...], w_ref[...],
                         preferred_element_type=jnp.float32)


def _prep_body(degp_ref, xw_ref, y_ref, dinv_ref):
    deg = degp_ref[0] + degp_ref[1] + 1.0
    dinv = lax.rsqrt(deg)
    dinv_ref[...] = dinv
    y_ref[...] = dinv * xw_ref[...]


def _mid_body(parts_ref, y_ref, dinv_ref, y2_ref):
    s = parts_ref[0] + parts_ref[1] - y_ref[...]
    d = dinv_ref[...]
    y2_ref[...] = (d * d) * s


def _final_body(parts_ref, y2_ref, dinv_ref, w2_ref, o_ref):
    s = parts_ref[0] + parts_ref[1] - y2_ref[...]
    g = dinv_ref[...] * s
    o_ref[...] = jnp.dot(g, w2_ref[...], preferred_element_type=jnp.float32)


def kernel(x, edge_index, W1, W2):
    f32 = jnp.float32
    x_pad = jnp.zeros((NPAD, D_IN), f32).at[:N].set(x)

    pad_cnt = E_PAD - E
    pad_idx = (N + jnp.arange(pad_cnt, dtype=jnp.int32) % (NPAD - N))
    src = jnp.concatenate([edge_index[0], pad_idx]).reshape(NW, CH, CHUNK)
    dst = jnp.concatenate([edge_index[1], pad_idx]).reshape(NW, CH, CHUNK)

    deg_parts = _sc_degree(dst)
    deg_parts = deg_parts.reshape(NC, NPAD, 1)

    xw1 = pl.pallas_call(
        _mm_body,
        out_shape=jax.ShapeDtypeStruct((NPAD, D_HID), f32),
    )(x_pad, W1)

    y1, dinv = pl.pallas_call(
        _prep_body,
        out_shape=(jax.ShapeDtypeStruct((NPAD, D_HID), f32),
                   jax.ShapeDtypeStruct((NPAD, 1), f32)),
    )(deg_parts, xw1)

    parts1 = _sc_aggregate(src, dst, y1)

    y2 = pl.pallas_call(
        _mid_body,
        out_shape=jax.ShapeDtypeStruct((NPAD, D_HID), f32),
    )(parts1, y1, dinv)

    parts2 = _sc_aggregate(src, dst, y2)

    out_pad = pl.pallas_call(
        _final_body,
        out_shape=jax.ShapeDtypeStruct((NPAD, D_OUT), f32),
    )(parts2, y2, dinv, W2)

    return out_pad[:N]


# speedup vs baseline: 61.4052x; 1.7133x over previous
import functools

import jax
import jax.numpy as jnp
from jax import lax
from jax.experimental import pallas as pl
from jax.experimental.pallas import tpu as pltpu
from jax.experimental.pallas import tpu_sc as plsc

N = 10000
E = 320000
D_IN = 128
D_HID = 16
D_OUT = 40

NC = 2
NS = 16
NW = NC * NS

NPAD = 10240
RPT = NPAD // NS
CHUNK = 128
K = 8
CH = 80
EPW = CH * CHUNK
E_PAD = EPW * NW

_mesh = plsc.VectorSubcoreMesh(
    core_axis_name="c", subcore_axis_name="s", num_cores=NC, num_subcores=NS)
_sc_params = pltpu.CompilerParams(use_tc_tiling_on_sc=False)


@functools.partial(
    pl.kernel,
    out_type=jax.ShapeDtypeStruct((NC, NPAD), jnp.float32),
    mesh=_mesh,
    scratch_types=[
        pltpu.VMEM((CH, CHUNK), jnp.int32),
        pltpu.VMEM((CHUNK,), jnp.float32),
        pltpu.VMEM((RPT,), jnp.float32),
        pltpu.VMEM_SHARED((NPAD,), jnp.float32),
    ],
    compiler_params=_sc_params,
)
def _sc_degree(dst_hbm, out_hbm, idx_v, ones_v, zeros_v, table):
    cid = lax.axis_index("c")
    sid = lax.axis_index("s")
    w = sid * NC + cid

    def fill_ones(i, _):
        ones_v[pl.ds(i * 16, 16)] = jnp.ones((16,), jnp.float32)
        return 0

    lax.fori_loop(0, CHUNK // 16, fill_ones, 0)

    def fill_zeros(i, _):
        zeros_v[pl.ds(i * 16, 16)] = jnp.zeros((16,), jnp.float32)
        return 0

    lax.fori_loop(0, RPT // 16, fill_zeros, 0)

    pltpu.sync_copy(zeros_v, table.at[pl.ds(sid * RPT, RPT)])
    pltpu.sync_copy(dst_hbm.at[w], idx_v)
    plsc.subcore_barrier()

    def body(j, _):
        pltpu.sync_copy(ones_v, table.at[idx_v.at[j]], add=True)
        return 0

    lax.fori_loop(0, CH, body, 0)
    plsc.subcore_barrier()
    pltpu.sync_copy(table.at[pl.ds(sid * RPT, RPT)],
                    out_hbm.at[cid, pl.ds(sid * RPT, RPT)])


@functools.partial(
    pl.kernel,
    out_type=jax.ShapeDtypeStruct((NC, NPAD, D_HID), jnp.float32),
    mesh=_mesh,
    scratch_types=[
        pltpu.VMEM((CH, CHUNK), jnp.int32),
        pltpu.VMEM((CH, CHUNK), jnp.int32),
        pltpu.VMEM((K, CHUNK, D_HID), jnp.float32),
        pltpu.VMEM_SHARED((NPAD, D_HID), jnp.float32),
        pltpu.SemaphoreType.DMA((K,)),
    ],
    compiler_params=_sc_params,
)
def _sc_aggregate(src_hbm, dst_hbm, y_hbm, out_hbm,
                  src_v, dst_v, rows_v, table, sems):
    cid = lax.axis_index("c")
    sid = lax.axis_index("s")
    w = sid * NC + cid

    pltpu.sync_copy(y_hbm.at[pl.ds(sid * RPT, RPT)],
                    table.at[pl.ds(sid * RPT, RPT)])
    pltpu.sync_copy(src_hbm.at[w], src_v)
    pltpu.sync_copy(dst_hbm.at[w], dst_v)
    plsc.subcore_barrier()

    for b in range(K):
        pltpu.async_copy(y_hbm.at[src_v.at[b]], rows_v.at[b], sems.at[b])

    def outer(jb, _):
        for b in range(K):
            j = jb * K + b
            pltpu.make_async_copy(
                y_hbm.at[src_v.at[j]], rows_v.at[b], sems.at[b]).wait()
            pltpu.sync_copy(rows_v.at[b], table.at[dst_v.at[j]], add=True)
            nj = j + K

            @pl.when(nj < CH)
            def _():
                pltpu.async_copy(
                    y_hbm.at[src_v.at[nj]], rows_v.at[b], sems.at[b])
        return 0

    lax.fori_loop(0, CH // K, outer, 0)
    plsc.subcore_barrier()
    pltpu.sync_copy(table.at[pl.ds(sid * RPT, RPT)],
                    out_hbm.at[cid, pl.ds(sid * RPT, RPT)])


def _mm_body(x_ref, w_ref, o_ref):
    o_ref[...] = jnp.dot(x_ref[...], w_ref[...],
                         preferred_element_type=jnp.float32)


def _prep_body(degp_ref, xw_ref, y_ref, dinv_ref):
    deg = degp_ref[0] + degp_ref[1] + 1.0
    dinv = lax.rsqrt(deg)
    dinv_ref[...] = dinv
    y_ref[...] = dinv * xw_ref[...]


def _mid_body(parts_ref, y_ref, dinv_ref, y2_ref):
    s = parts_ref[0] + parts_ref[1] - y_ref[...]
    d = dinv_ref[...]
    y2_ref[...] = (d * d) * s


def _final_body(parts_ref, y2_ref, dinv_ref, w2_ref, o_ref):
    s = parts_ref[0] + parts_ref[1] - y2_ref[...]
    g = dinv_ref[...] * s
    o_ref[...] = jnp.dot(g, w2_ref[...], preferred_element_type=jnp.float32)


def kernel(x, edge_index, W1, W2):
    f32 = jnp.float32
    x_pad = jnp.zeros((NPAD, D_IN), f32).at[:N].set(x)

    pad_cnt = E_PAD - E
    pad_idx = (N + jnp.arange(pad_cnt, dtype=jnp.int32) % (NPAD - N))
    src = jnp.concatenate([edge_index[0], pad_idx]).reshape(NW, CH, CHUNK)
    dst = jnp.concatenate([edge_index[1], pad_idx]).reshape(NW, CH, CHUNK)

    deg_parts = _sc_degree(dst)
    deg_parts = deg_parts.reshape(NC, NPAD, 1)

    xw1 = pl.pallas_call(
        _mm_body,
        out_shape=jax.ShapeDtypeStruct((NPAD, D_HID), f32),
    )(x_pad, W1)

    y1, dinv = pl.pallas_call(
        _prep_body,
        out_shape=(jax.ShapeDtypeStruct((NPAD, D_HID), f32),
                   jax.ShapeDtypeStruct((NPAD, 1), f32)),
    )(deg_parts, xw1)

    parts1 = _sc_aggregate(src, dst, y1)

    y2 = pl.pallas_call(
        _mid_body,
        out_shape=jax.ShapeDtypeStruct((NPAD, D_HID), f32),
    )(parts1, y1, dinv)

    parts2 = _sc_aggregate(src, dst, y2)

    out_pad = pl.pallas_call(
        _final_body,
        out_shape=jax.ShapeDtypeStruct((NPAD, D_OUT), f32),
    )(parts2, y2, dinv, W2)

    return out_pad[:N]
